# baseline (speedup 1.0000x reference)
import functools

import jax
import jax.numpy as jnp
from jax import lax
from jax.experimental import pallas as pl
from jax.experimental.pallas import tpu as pltpu
from jax.experimental.pallas import tpu_sc as plsc

D = 64
SCALE = 8.0
NC = 2
NS = 16
NW = NC * NS
L = 16
TR = 16


def _sc_embed(tokens, table):
    R, S = tokens.shape
    r_per_w = R // NW
    n_chunks = r_per_w // TR
    mesh = plsc.VectorSubcoreMesh(core_axis_name="c", subcore_axis_name="s")

    @functools.partial(
        pl.kernel,
        mesh=mesh,
        out_type=jax.ShapeDtypeStruct((R, S, D), jnp.float32),
        scratch_types=[
            pltpu.VMEM((r_per_w, S), jnp.int32),
            pltpu.VMEM((TR, S, D), jnp.float32),
            pltpu.VMEM((TR, S, D), jnp.float32),
            pltpu.SemaphoreType.DMA,
            pltpu.SemaphoreType.DMA,
            pltpu.SemaphoreType.DMA,
            pltpu.SemaphoreType.DMA,
        ],
        compiler_params=pltpu.CompilerParams(use_tc_tiling_on_sc=False),
    )
    def k(table_hbm, tok_hbm, out_hbm, idx_all, rows0, rows1, g0, g1, w0, w1):
        wid = lax.axis_index("s") * NC + lax.axis_index("c")
        base = wid * r_per_w
        rows = (rows0, rows1)
        gsem = (g0, g1)
        wsem = (w0, w1)

        pltpu.sync_copy(tok_hbm.at[pl.ds(base, r_per_w)], idx_all)

        def gather(j, buf):
            return [
                pltpu.async_copy(
                    table_hbm.at[idx_all.at[j * TR + i]],
                    rows[buf].at[i], gsem[buf])
                for i in range(TR)
            ]

        def writeback(j, buf):
            return pltpu.async_copy(
                rows[buf], out_hbm.at[pl.ds(base + j * TR, TR)],
                wsem[buf])

        g_handles = [None, None]
        w_handles = [None, None]
        g_handles[0] = gather(0, 0)
        for j in range(n_chunks):
            cur = j & 1
            nxt = cur ^ 1
            if j + 1 < n_chunks:
                if w_handles[nxt] is not None:
                    w_handles[nxt].wait()
                g_handles[nxt] = gather(j + 1, nxt)
            for h in g_handles[cur]:
                h.wait()

            r = rows[cur]

            @plsc.parallel_loop(0, S, 1, unroll=2)
            def _scale(col):
                for i in range(TR):
                    for q in range(D // L):
                        r[i, col, pl.ds(q * L, L)] = (
                            r[i, col, pl.ds(q * L, L)] * SCALE)

            w_handles[cur] = writeback(j, cur)
        w_handles[0].wait()
        w_handles[1].wait()

    return k(table, tokens)


def kernel(tokens, table):
    return _sc_embed(tokens, table)


# speedup vs baseline: 1.0088x; 1.0088x over previous
import functools

import jax
import jax.numpy as jnp
from jax import lax
from jax.experimental import pallas as pl
from jax.experimental.pallas import tpu as pltpu
from jax.experimental.pallas import tpu_sc as plsc

D = 64
SCALE = 8.0
NC = 2
NS = 16
NW = NC * NS
L = 16
CH = 800


def _sc_embed(flat_tok, table):
    N, = flat_tok.shape
    n_per_w = N // NW
    n_chunks = n_per_w // CH
    mesh = plsc.VectorSubcoreMesh(core_axis_name="c", subcore_axis_name="s")

    @functools.partial(
        pl.kernel,
        mesh=mesh,
        out_type=jax.ShapeDtypeStruct((N, D), jnp.float32),
        scratch_types=[
            pltpu.VMEM((n_per_w,), jnp.int32),
            pltpu.VMEM((CH, D), jnp.float32),
            pltpu.VMEM((CH, D), jnp.float32),
            pltpu.SemaphoreType.DMA,
            pltpu.SemaphoreType.DMA,
            pltpu.SemaphoreType.DMA,
            pltpu.SemaphoreType.DMA,
        ],
        compiler_params=pltpu.CompilerParams(use_tc_tiling_on_sc=False),
    )
    def k(table_hbm, tok_hbm, out_hbm, idx_all, rows0, rows1, g0, g1, w0, w1):
        wid = lax.axis_index("s") * NC + lax.axis_index("c")
        base = wid * n_per_w
        rows = (rows0, rows1)
        gsem = (g0, g1)
        wsem = (w0, w1)

        pltpu.sync_copy(tok_hbm.at[pl.ds(base, n_per_w)], idx_all)

        def gather(j, buf):
            return pltpu.async_copy(
                table_hbm.at[idx_all.at[pl.ds(j * CH, CH)]],
                rows[buf], gsem[buf])

        def writeback(j, buf):
            return pltpu.async_copy(
                rows[buf], out_hbm.at[pl.ds(base + j * CH, CH)],
                wsem[buf])

        g_handles = [None, None]
        w_handles = [None, None]
        g_handles[0] = gather(0, 0)
        for j in range(n_chunks):
            cur = j & 1
            nxt = cur ^ 1
            if j + 1 < n_chunks:
                if w_handles[nxt] is not None:
                    w_handles[nxt].wait()
                g_handles[nxt] = gather(j + 1, nxt)
            g_handles[cur].wait()

            r = rows[cur]

            @plsc.parallel_loop(0, CH, 1, unroll=4)
            def _scale(i):
                for q in range(D // L):
                    r[i, pl.ds(q * L, L)] = r[i, pl.ds(q * L, L)] * SCALE

            w_handles[cur] = writeback(j, cur)
        w_handles[0].wait()
        w_handles[1].wait()

    return k(table, flat_tok)


def kernel(tokens, table):
    R, S = tokens.shape
    out = _sc_embed(tokens.reshape(R * S), table)
    return out.reshape(R, S, D)


# speedup vs baseline: 1.1888x; 1.1784x over previous
import functools

import jax
import jax.numpy as jnp
from jax import lax
from jax.experimental import pallas as pl
from jax.experimental.pallas import tpu as pltpu
from jax.experimental.pallas import tpu_sc as plsc

D = 64
SCALE = 8.0
NC = 2
NS = 16
NW = NC * NS
L = 16
TR = 4


def _sc_embed(idx2, par, table2):
    R, S = idx2.shape
    r_per_w = R // NW
    n_chunks = r_per_w // TR
    n_pairs = n_chunks // 2
    mesh = plsc.VectorSubcoreMesh(core_axis_name="c", subcore_axis_name="s")

    @functools.partial(
        pl.kernel,
        mesh=mesh,
        out_type=jax.ShapeDtypeStruct((R, S, 2 * D), jnp.float32),
        scratch_types=[
            pltpu.VMEM((r_per_w, S), jnp.int32),
            pltpu.VMEM((r_per_w, S), jnp.int32),
            pltpu.VMEM((TR, S, 2 * D), jnp.float32),
            pltpu.VMEM((TR, S, 2 * D), jnp.float32),
            pltpu.SemaphoreType.DMA,
            pltpu.SemaphoreType.DMA,
            pltpu.SemaphoreType.DMA,
            pltpu.SemaphoreType.DMA,
        ],
        compiler_params=pltpu.CompilerParams(use_tc_tiling_on_sc=True),
    )
    def k(table_hbm, idx_hbm, par_hbm, out_hbm,
          idx_all, par_all, rows0, rows1, g0, g1, w0, w1):
        wid = lax.axis_index("s") * NC + lax.axis_index("c")
        base = wid * r_per_w
        rows = (rows0, rows1)
        gsem = (g0, g1)
        wsem = (w0, w1)

        pltpu.sync_copy(idx_hbm.at[pl.ds(base, r_per_w)], idx_all)
        pltpu.sync_copy(par_hbm.at[pl.ds(base, r_per_w)], par_all)

        def g_issue(c, b):
            for i in range(TR):
                pltpu.async_copy(
                    table_hbm.at[idx_all.at[c * TR + i]],
                    rows[b].at[i], gsem[b])

        def g_wait(c, b):
            for i in range(TR):
                pltpu.make_async_copy(
                    table_hbm.at[idx_all.at[c * TR + i]],
                    rows[b].at[i], gsem[b]).wait()

        def wb_issue(c, b):
            pltpu.async_copy(
                rows[b], out_hbm.at[pl.ds(base + c * TR, TR)], wsem[b])

        def wb_wait(c, b):
            pltpu.make_async_copy(
                rows[b], out_hbm.at[pl.ds(base + c * TR, TR)],
                wsem[b]).wait()

        def select(c, b):
            r = rows[b]

            @plsc.parallel_loop(0, S, 1, unroll=2)
            def _sel(col):
                for i in range(TR):
                    pv = par_all[c * TR + i, pl.ds(col, 1)]
                    off = pv[0] * D
                    for q in range(D // L):
                        r[i, col, pl.ds(q * L, L)] = (
                            r[i, col, pl.ds(off + q * L, L)] * SCALE)

        g_issue(0, 0)

        def body(j, carry):
            c0 = 2 * j

            @pl.when(j > 0)
            def _():
                wb_wait(c0 - 1, 1)

            g_issue(c0 + 1, 1)
            g_wait(c0, 0)
            select(c0, 0)
            wb_issue(c0, 0)

            @pl.when(j < n_pairs - 1)
            def _():
                wb_wait(c0, 0)
                g_issue(c0 + 2, 0)

            g_wait(c0 + 1, 1)
            select(c0 + 1, 1)
            wb_issue(c0 + 1, 1)
            return carry

        lax.fori_loop(0, n_pairs, body, 0)
        wb_wait(n_chunks - 2, 0)
        wb_wait(n_chunks - 1, 1)

    return k(table2, idx2, par)


def kernel(tokens, table):
    idx2 = jax.lax.shift_right_logical(tokens, 1)
    par = jnp.bitwise_and(tokens, 1)
    table2 = table.reshape(table.shape[0] // 2, 2 * table.shape[1])
    out2 = _sc_embed(idx2, par, table2)
    return out2[:, :, :D]


# speedup vs baseline: 1.2837x; 1.0798x over previous
import functools

import jax
import jax.numpy as jnp
from jax import lax
from jax.experimental import pallas as pl
from jax.experimental.pallas import tpu as pltpu
from jax.experimental.pallas import tpu_sc as plsc

D = 64
SCALE = 8.0
NC = 2
NS = 16
NW = NC * NS
L = 16
TR = 4


def _sc_embed(tokens, tableP):
    R, S = tokens.shape
    r_per_w = R // NW
    n_chunks = r_per_w // TR
    n_pairs = n_chunks // 2
    mesh = plsc.VectorSubcoreMesh(core_axis_name="c", subcore_axis_name="s")

    @functools.partial(
        pl.kernel,
        mesh=mesh,
        out_type=jax.ShapeDtypeStruct((R, S, 2 * D), jnp.float32),
        scratch_types=[
            pltpu.VMEM((r_per_w, S), jnp.int32),
            pltpu.VMEM((TR, S, 2 * D), jnp.float32),
            pltpu.VMEM((TR, S, 2 * D), jnp.float32),
            pltpu.SemaphoreType.DMA,
            pltpu.SemaphoreType.DMA,
            pltpu.SemaphoreType.DMA,
            pltpu.SemaphoreType.DMA,
        ],
        compiler_params=pltpu.CompilerParams(use_tc_tiling_on_sc=True),
    )
    def k(table_hbm, tok_hbm, out_hbm,
          idx_all, rows0, rows1, g0, g1, w0, w1):
        wid = lax.axis_index("s") * NC + lax.axis_index("c")
        base = wid * r_per_w
        rows = (rows0, rows1)
        gsem = (g0, g1)
        wsem = (w0, w1)

        pltpu.sync_copy(tok_hbm.at[pl.ds(base, r_per_w)], idx_all)

        def g_issue(c, b):
            for i in range(TR):
                pltpu.async_copy(
                    table_hbm.at[idx_all.at[c * TR + i]],
                    rows[b].at[i], gsem[b])

        def g_wait(c, b):
            for i in range(TR):
                pltpu.make_async_copy(
                    table_hbm.at[idx_all.at[c * TR + i]],
                    rows[b].at[i], gsem[b]).wait()

        def wb_issue(c, b):
            pltpu.async_copy(
                rows[b], out_hbm.at[pl.ds(base + c * TR, TR)], wsem[b])

        def wb_wait(c, b):
            pltpu.make_async_copy(
                rows[b], out_hbm.at[pl.ds(base + c * TR, TR)],
                wsem[b]).wait()

        def scale(b):
            r = rows[b]

            @plsc.parallel_loop(0, S, 1, unroll=2)
            def _sc(col):
                for i in range(TR):
                    for q in range(D // L):
                        r[i, col, pl.ds(q * L, L)] = (
                            r[i, col, pl.ds(q * L, L)] * SCALE)

        g_issue(0, 0)

        def body(j, carry):
            c0 = 2 * j

            @pl.when(j > 0)
            def _():
                wb_wait(c0 - 1, 1)

            g_issue(c0 + 1, 1)
            g_wait(c0, 0)
            scale(0)
            wb_issue(c0, 0)

            @pl.when(j < n_pairs - 1)
            def _():
                wb_wait(c0, 0)
                g_issue(c0 + 2, 0)

            g_wait(c0 + 1, 1)
            scale(1)
            wb_issue(c0 + 1, 1)
            return carry

        lax.fori_loop(0, n_pairs, body, 0)
        wb_wait(n_chunks - 2, 0)
        wb_wait(n_chunks - 1, 1)

    return k(tableP, tokens)


def kernel(tokens, table):
    tableP = jnp.pad(table, ((0, 0), (0, D)))
    out2 = _sc_embed(tokens, tableP)
    return out2[:, :, :D]
